# baseline (speedup 1.0000x reference)
import functools

import numpy as np

import jax
import jax.numpy as jnp
from jax import lax
from jax.experimental import pallas as pl
from jax.experimental.pallas import tpu as pltpu
from jax.experimental.pallas import tpu_sc as plsc

_F = 64
_NSPECIES = 14
_DIMS = (1, 3, 5)
_W = 640
_NROW = 16


def _lut_pack(leq0, leq1, leq2):
    lut = jnp.concatenate(
        [leq.reshape(_NSPECIES, _F * k) for leq, k in
         zip((leq0, leq1, leq2), _DIMS)], axis=1)
    lut = jnp.pad(lut.T, ((0, _W - lut.shape[1]), (0, _NROW - _NSPECIES)))
    return lut.reshape(-1)


def _gather3(z, lut):
    n = z.shape[0]
    info = plsc.get_sparse_core_info()
    nc, ns = info.num_cores, info.num_subcores
    nw = nc * ns
    bw = n // nw
    nbl = bw // 128
    nblocks = n // 128

    @functools.partial(
        pl.kernel,
        mesh=plsc.VectorSubcoreMesh(core_axis_name="c", subcore_axis_name="s"),
        compiler_params=pltpu.CompilerParams(
            use_tc_tiling_on_sc=False, needs_layout_passes=False),
        out_type=[
            jax.ShapeDtypeStruct((_F, 1, n), jnp.float32),
            jax.ShapeDtypeStruct((3, _F // 8, nblocks, 8, 128), jnp.float32),
            jax.ShapeDtypeStruct((5, _F // 8, nblocks, 8, 128), jnp.float32),
        ],
        scratch_types=[
            pltpu.VMEM((bw,), jnp.int32),
            pltpu.VMEM((_NROW * _W,), jnp.float32),
            pltpu.VMEM((2, 8, 8, 128), jnp.float32),
            pltpu.VMEM((2, 8, 1, 1024), jnp.float32),
            pltpu.SemaphoreType.DMA,
            pltpu.SemaphoreType.DMA,
        ],
    )
    def k(z_hbm, lut_hbm, o0_hbm, o1_hbm, o2_hbm, zv, lutv, stg, stg0,
          sem0, sem1):
        wid = lax.axis_index("s") * nc + lax.axis_index("c")
        nb0 = wid * nbl
        pltpu.sync_copy(z_hbm.at[pl.ds(wid * bw, bw)], zv)
        pltpu.sync_copy(lut_hbm, lutv)
        sems = (sem0, sem1)

        def compute(buf, col0, cstride, f_major):
            def blk(nb, carry):
                zvecs = [zv[pl.ds(nb * 128 + l * 16, 16)] for l in range(8)]
                for l in range(8):
                    vals = []
                    for fi in range(8):
                        sub = lutv.at[pl.ds((col0 + fi * cstride) * _NROW,
                                            _NROW)]
                        vals.append(plsc.load_gather(sub, [zvecs[l]]))
                    for fi, v in enumerate(vals):
                        if f_major:
                            stg0[buf, fi, 0, pl.ds(nb * 128 + l * 16, 16)] = v
                        else:
                            stg[buf, nb, fi, pl.ds(l * 16, 16)] = v
                return carry

            lax.fori_loop(0, nbl, blk, 0)

        def section(nloop, out_dst, col_of, cstride, f_major):
            buf = stg0 if f_major else stg

            def body2(g, carry):
                for par in range(2):
                    i = 2 * g + par

                    @pl.when(i >= 2)
                    def _():
                        pltpu.make_async_copy(
                            buf.at[par], out_dst(i - 2), sems[par]).wait()

                    compute(par, col_of(i), cstride, f_major)
                    pltpu.async_copy(buf.at[par], out_dst(i), sems[par])
                return carry

            lax.fori_loop(0, nloop // 2, body2, 0)
            for par in range(2):
                i = nloop - 2 + par
                pltpu.make_async_copy(buf.at[par], out_dst(i), sems[par]).wait()

        section(
            8,
            lambda i: o0_hbm.at[pl.ds(i * 8, 8), pl.ds(0, 1),
                                pl.ds(wid * bw, bw)],
            lambda i: i * 8,
            1,
            True,
        )
        for j in range(3):
            section(
                8,
                lambda i, j=j: o1_hbm.at[j, i, pl.ds(nb0, nbl)],
                lambda i, j=j: _F + i * 8 * 3 + j,
                3,
                False,
            )
        for j in range(5):
            section(
                8,
                lambda i, j=j: o2_hbm.at[j, i, pl.ds(nb0, nbl)],
                lambda i, j=j: _F * 4 + i * 8 * 5 + j,
                5,
                False,
            )

    return k(z, lut)


def kernel(Z, leq0, leq1, leq2):
    N = Z.shape[0]
    z = Z.astype(jnp.int32)
    lut = _lut_pack(leq0, leq1, leq2)
    o0, o1, o2 = _gather3(z, lut)
    out0 = jnp.transpose(o0, (2, 0, 1))
    out1 = jnp.transpose(o1, (2, 4, 1, 3, 0)).reshape(N, _F, 3)
    out2 = jnp.transpose(o2, (2, 4, 1, 3, 0)).reshape(N, _F, 5)
    return (
        out0,
        out1,
        out2,
        _zeros(N, 7),
        _zeros(N, 9),
        _zeros(N, 11),
    )


@functools.lru_cache(maxsize=None)
def _zeros(n, k):
    return np.zeros((n, _F, k), np.float32)


# speedup vs baseline: 1.0295x; 1.0295x over previous
import functools

import jax
import jax.numpy as jnp
from jax import lax
from jax.experimental import pallas as pl
from jax.experimental.pallas import tpu as pltpu
from jax.experimental.pallas import tpu_sc as plsc

_F = 64
_NSPECIES = 14
_DIMS = (1, 3, 5)
_W = 640
_NROW = 16


def _lut_pack(leq0, leq1, leq2):
    lut = jnp.concatenate(
        [leq.reshape(_NSPECIES, _F * k) for leq, k in
         zip((leq0, leq1, leq2), _DIMS)], axis=1)
    lut = jnp.pad(lut.T, ((0, _W - lut.shape[1]), (0, _NROW - _NSPECIES)))
    return lut.reshape(-1)


def _gather3(z, lut):
    n = z.shape[0]
    info = plsc.get_sparse_core_info()
    nc, ns = info.num_cores, info.num_subcores
    nw = nc * ns
    bw = n // nw
    nbl = bw // 128
    nblocks = n // 128

    @functools.partial(
        pl.kernel,
        mesh=plsc.VectorSubcoreMesh(core_axis_name="c", subcore_axis_name="s"),
        compiler_params=pltpu.CompilerParams(
            use_tc_tiling_on_sc=False, needs_layout_passes=False),
        out_type=[
            jax.ShapeDtypeStruct((_F, 1, n), jnp.float32),
            jax.ShapeDtypeStruct((3, _F // 8, nblocks, 8, 128), jnp.float32),
            jax.ShapeDtypeStruct((5, _F // 8, nblocks, 8, 128), jnp.float32),
            jax.ShapeDtypeStruct((n * _F * 9 // 128, 128), jnp.float32),
        ],
        scratch_types=[
            pltpu.VMEM((bw,), jnp.int32),
            pltpu.VMEM((_NROW * _W,), jnp.float32),
            pltpu.VMEM((2, 8, 8, 128), jnp.float32),
            pltpu.VMEM((2, 8, 1, 1024), jnp.float32),
            pltpu.VMEM((64, 128), jnp.float32),
            pltpu.SemaphoreType.DMA,
            pltpu.SemaphoreType.DMA,
            pltpu.SemaphoreType.DMA,
        ],
    )
    def k(z_hbm, lut_hbm, o0_hbm, o1_hbm, o2_hbm, o4_hbm, zv, lutv, stg,
          stg0, zbuf, sem0, sem1, semz):
        wid = lax.axis_index("s") * nc + lax.axis_index("c")
        nb0 = wid * nbl
        pltpu.sync_copy(z_hbm.at[pl.ds(wid * bw, bw)], zv)
        pltpu.sync_copy(lut_hbm, lutv)
        sems = (sem0, sem1)

        zrows = o4_hbm.shape[0] // nw
        nzc = zrows // 64
        for r in range(64):
            for c in range(8):
                zbuf[r, pl.ds(c * 16, 16)] = jnp.zeros((16,), jnp.float32)

        def zfire(i, carry):
            pltpu.async_copy(
                zbuf, o4_hbm.at[pl.ds(wid * zrows + i * 64, 64)], semz)
            return carry

        lax.fori_loop(0, nzc, zfire, 0)

        def compute(buf, col0, cstride, f_major):
            def blk(nb, carry):
                zvecs = [zv[pl.ds(nb * 128 + l * 16, 16)] for l in range(8)]
                for l in range(8):
                    vals = []
                    for fi in range(8):
                        sub = lutv.at[pl.ds((col0 + fi * cstride) * _NROW,
                                            _NROW)]
                        vals.append(plsc.load_gather(sub, [zvecs[l]]))
                    for fi, v in enumerate(vals):
                        if f_major:
                            stg0[buf, fi, 0, pl.ds(nb * 128 + l * 16, 16)] = v
                        else:
                            stg[buf, nb, fi, pl.ds(l * 16, 16)] = v
                return carry

            lax.fori_loop(0, nbl, blk, 0)

        def section(nloop, out_dst, col_of, cstride, f_major):
            buf = stg0 if f_major else stg

            def body2(g, carry):
                for par in range(2):
                    i = 2 * g + par

                    @pl.when(i >= 2)
                    def _():
                        pltpu.make_async_copy(
                            buf.at[par], out_dst(i - 2), sems[par]).wait()

                    compute(par, col_of(i), cstride, f_major)
                    pltpu.async_copy(buf.at[par], out_dst(i), sems[par])
                return carry

            lax.fori_loop(0, nloop // 2, body2, 0)
            for par in range(2):
                i = nloop - 2 + par
                pltpu.make_async_copy(buf.at[par], out_dst(i), sems[par]).wait()

        section(
            8,
            lambda i: o0_hbm.at[pl.ds(i * 8, 8), pl.ds(0, 1),
                                pl.ds(wid * bw, bw)],
            lambda i: i * 8,
            1,
            True,
        )
        for j in range(3):
            section(
                8,
                lambda i, j=j: o1_hbm.at[j, i, pl.ds(nb0, nbl)],
                lambda i, j=j: _F + i * 8 * 3 + j,
                3,
                False,
            )
        for j in range(5):
            section(
                8,
                lambda i, j=j: o2_hbm.at[j, i, pl.ds(nb0, nbl)],
                lambda i, j=j: _F * 4 + i * 8 * 5 + j,
                5,
                False,
            )

        def zdrain(i, carry):
            pltpu.make_async_copy(
                zbuf, o4_hbm.at[pl.ds(wid * zrows + i * 64, 64)],
                semz).wait()
            return carry

        lax.fori_loop(0, nzc, zdrain, 0)

    return k(z, lut)


def kernel(Z, leq0, leq1, leq2):
    N = Z.shape[0]
    z = Z.astype(jnp.int32)
    lut = _lut_pack(leq0, leq1, leq2)
    o0, o1, o2, o4 = _gather3(z, lut)
    out0 = jnp.transpose(o0, (2, 0, 1))
    out1 = jnp.transpose(o1, (2, 4, 1, 3, 0)).reshape(N, _F, 3)
    out2 = jnp.transpose(o2, (2, 4, 1, 3, 0)).reshape(N, _F, 5)
    return (
        out0,
        out1,
        out2,
        jnp.zeros((N, _F, 7), jnp.float32),
        jnp.transpose(o4.reshape(9, 8, N // 128, 8, 128),
                      (2, 4, 1, 3, 0)).reshape(N, _F, 9),
        jnp.zeros((N, _F, 11), jnp.float32),
    )
